# baseline (speedup 1.0000x reference)
import functools

import jax
import jax.numpy as jnp
from jax import lax
from jax.experimental import pallas as pl
from jax.experimental.pallas import tpu as pltpu
from jax.experimental.pallas import tpu_sc as plsc

B = 16384
D = 64
HW = 32
NC = 2
NS = 16
NW = NC * NS
BPW = B // NW
L = 16
SCALE = 0.25


@functools.partial(
    pl.kernel,
    out_type=(
        jax.ShapeDtypeStruct((B * D,), jnp.float32),
        jax.ShapeDtypeStruct((B * D,), jnp.float32),
    ),
    mesh=plsc.VectorSubcoreMesh(core_axis_name="c", subcore_axis_name="s"),
    scratch_types=[
        pltpu.VMEM((BPW,), jnp.int32),
        pltpu.VMEM((2 * BPW,), jnp.int32),
        pltpu.VMEM((2 * BPW, HW), jnp.float32),
        pltpu.VMEM((BPW * D,), jnp.float32),
        pltpu.SemaphoreType.DMA,
    ],
    compiler_params=pltpu.CompilerParams(use_tc_tiling_on_sc=False, needs_layout_passes=False),
)
def _gather_scale(users_hbm, items_hbm, utab_hbm, itab_hbm,
                  out_u_hbm, out_i_hbm,
                  idx_v, pidx_v, rows_v, out_v, sem):
    wid = lax.axis_index("s") * NC + lax.axis_index("c")
    base = wid * BPW
    lane = lax.iota(jnp.int32, L)

    def one_table(src_idx_hbm, tab_hbm, out_hbm):
        pltpu.sync_copy(src_idx_hbm.at[pl.ds(base, BPW)], idx_v)

        def build(t, _):
            iv = idx_v[pl.ds(t * L, L)]
            dv = iv * 2
            pos = (t * L + lane) * 2
            plsc.store_scatter(pidx_v, [pos], dv)
            plsc.store_scatter(pidx_v, [pos + 1], dv + 1)
            return 0

        lax.fori_loop(0, BPW // L, build, 0, unroll=4)
        pltpu.async_copy(tab_hbm.at[pidx_v], rows_v, sem).wait()

        def scale(i, _):
            out_v[pl.ds(i * HW, L)] = rows_v[i, pl.ds(0, L)] * SCALE
            out_v[pl.ds(i * HW + L, L)] = rows_v[i, pl.ds(L, L)] * SCALE
            return 0

        lax.fori_loop(0, 2 * BPW, scale, 0, unroll=8)
        pltpu.sync_copy(out_v, out_hbm.at[pl.ds(base * D, BPW * D)])

    one_table(users_hbm, utab_hbm, out_u_hbm)
    one_table(items_hbm, itab_hbm, out_i_hbm)


def kernel(users, items, user_table, item_table, edge_user, edge_item):
    del edge_user, edge_item
    utab32 = user_table.reshape(-1, HW)
    itab32 = item_table.reshape(-1, HW)
    out_u, out_i = _gather_scale(users, items, utab32, itab32)
    return (out_u.reshape(B, D), out_i.reshape(B, D))
